# Initial kernel scaffold; baseline (speedup 1.0000x reference)
import jax
import jax.numpy as jnp
from jax.experimental import pallas as pl


def kernel(x, W, b):
    raise NotImplementedError("write your pallas kernel here")


# speedup vs baseline: 1.3548x; 1.3548x over previous
import jax
import jax.numpy as jnp
from jax.experimental import pallas as pl
from jax.experimental.pallas import tpu as pltpu

E = 8
TOPK = 2
BLOCK = 1024


def _gate_kernel(x_ref, w_ref, b_ref, wout_ref, iout_ref):
    x = x_ref[...]
    w = w_ref[...]
    scores = jax.lax.dot_general(
        x, w, (((1,), (1,)), ((), ())),
        preferred_element_type=jnp.float32)
    scores = jnp.sqrt(jax.nn.softplus(scores))
    biased = scores + b_ref[...]

    idx = jax.lax.broadcasted_iota(jnp.int32, biased.shape, 1)
    m1 = jnp.max(biased, axis=-1, keepdims=True)
    i1 = jnp.min(jnp.where(biased == m1, idx, E), axis=-1, keepdims=True)
    masked = jnp.where(idx == i1, -jnp.inf, biased)
    m2 = jnp.max(masked, axis=-1, keepdims=True)
    i2 = jnp.min(jnp.where(masked == m2, idx, E), axis=-1, keepdims=True)

    w1 = jnp.sum(jnp.where(idx == i1, scores, 0.0), axis=-1, keepdims=True)
    w2 = jnp.sum(jnp.where(idx == i2, scores, 0.0), axis=-1, keepdims=True)
    s = w1 + w2
    wout_ref[...] = jnp.concatenate([w1 / s, w2 / s], axis=-1)
    iout_ref[...] = jnp.concatenate([i1, i2], axis=-1)


def kernel(x, W, b):
    T = x.shape[0]
    grid = (T // BLOCK,)
    weights, indices = pl.pallas_call(
        _gate_kernel,
        grid=grid,
        in_specs=[
            pl.BlockSpec((BLOCK, x.shape[1]), lambda i: (i, 0)),
            pl.BlockSpec((E, x.shape[1]), lambda i: (0, 0)),
            pl.BlockSpec((1, E), lambda i: (0, 0)),
        ],
        out_specs=[
            pl.BlockSpec((BLOCK, TOPK), lambda i: (i, 0)),
            pl.BlockSpec((BLOCK, TOPK), lambda i: (i, 0)),
        ],
        out_shape=[
            jax.ShapeDtypeStruct((T, TOPK), jnp.float32),
            jax.ShapeDtypeStruct((T, TOPK), jnp.int32),
        ],
    )(x, W, b.reshape(1, E))
    return (weights, indices)
